# baseline (speedup 1.0000x reference)
import functools

import numpy as np

import jax
import jax.numpy as jnp
from jax import lax
from jax.experimental import pallas as pl
from jax.experimental.pallas import tpu as pltpu
from jax.experimental.pallas import tpu_sc as plsc

_NB = 16384
_F = 26
_D = 64
_NC = 2
_NS = 16
_NW = _NC * _NS
_BC = 256
_NCH = _NB // _BC
_NJOB = _F * _NCH
_JPW = _NJOB // _NW


_COLC_NP = np.zeros((16, 4, 16), np.int32)
_STC_NP = np.zeros((16, 4, 16), np.int32)
for _m in range(16):
  for _k in range(4):
    for _l in range(16):
      _w = (_m + _l) & 15
      _COLC_NP[_m, _k, _l] = _k * 16 + _w
      _STC_NP[_m, _k, _l] = (2 * _k + _w // 8) * 2048 + (_w % 8) * 128 + _l
_COLC_2D = _COLC_NP.reshape(64, 16)
_STC_2D = _STC_NP.reshape(64, 16)


def _make_gather():
  mesh = plsc.VectorSubcoreMesh(core_axis_name="c", subcore_axis_name="s")

  @functools.partial(
      pl.kernel,
      out_type=jax.ShapeDtypeStruct((_F * 1048576,), jnp.float32),
      mesh=mesh,
      compiler_params=pltpu.CompilerParams(
          use_tc_tiling_on_sc=False, needs_layout_passes=False),
      scratch_types=[
          pltpu.VMEM((2, 128), jnp.int32),
          pltpu.VMEM((2, 128), jnp.int32),
          pltpu.VMEM((_BC, 128), jnp.float32),
          pltpu.VMEM((_BC, 128), jnp.float32),
          pltpu.VMEM((16384,), jnp.float32),
          pltpu.VMEM((16384,), jnp.float32),
          pltpu.VMEM((64, 16), jnp.int32),
          pltpu.VMEM((64, 16), jnp.int32),
          pltpu.SemaphoreType.DMA,
          pltpu.SemaphoreType.DMA,
          pltpu.SemaphoreType.DMA,
          pltpu.SemaphoreType.DMA,
      ],
  )
  def gather_kernel(idx_hbm, table_hbm, colc_hbm, stc_hbm, out_hbm, idx_v0,
                    idx_v1, rows_v0, rows_v1, tr_v0, tr_v1, colc_v, stc_v,
                    sem_g0, sem_g1, sem_s0, sem_s1):
    pltpu.sync_copy(colc_hbm, colc_v)
    pltpu.sync_copy(stc_hbm, stc_v)
    wid = lax.axis_index("s") * _NC + lax.axis_index("c")
    job0 = wid * _JPW
    idx_v = (idx_v0, idx_v1)
    rows = (rows_v0, rows_v1)
    tr = (tr_v0, tr_v1)
    sem_g = (sem_g0, sem_g1)
    sem_s = (sem_s0, sem_s1)

    def fire(j, b):
      f = j // _NCH
      c = lax.rem(j, _NCH)
      pltpu.sync_copy(idx_hbm.at[f, c], idx_v[b])
      for h in range(2):
        pltpu.async_copy(
            table_hbm.at[idx_v[b].at[h]],
            rows[b].at[pl.ds(h * 128, 128)],
            sem_g[b],
        )

    def wait_gathers(b):
      for h in range(2):
        pltpu.make_async_copy(
            table_hbm.at[idx_v[b].at[h]],
            rows[b].at[pl.ds(h * 128, 128)],
            sem_g[b],
        ).wait()

    def process(b):
      lanes = lax.iota(jnp.int32, 16)

      @plsc.parallel_loop(0, 16, unroll=4)
      def _trans(r16):
        rowvec = r16 * 16 + lanes
        sb = jnp.zeros((16,), jnp.int32) + ((r16 // 8) * 1024 + lax.rem(r16, 8) * 16)
        pend = []
        for kk in range(4):
          for m in range(16):
            ck = m * 4 + kk
            vals = plsc.load_gather(rows[b], [rowvec, colc_v[ck]])
            pend.append((ck, vals))
            if len(pend) >= 6:
              ck2, vals2 = pend.pop(0)
              plsc.store_scatter(tr[b], [sb + stc_v[ck2]], vals2)
        for ck2, vals2 in pend:
          plsc.store_scatter(tr[b], [sb + stc_v[ck2]], vals2)

    def fire_store(j, b):
      f = j // _NCH
      c = lax.rem(j, _NCH)
      for s0 in range(8):
        pltpu.async_copy(
            tr[b].at[pl.ds(s0 * 2048, 2048)],
            out_hbm.at[pl.ds(f * 1048576 + s0 * 131072 + c * 2048, 2048)],
            sem_s[b],
        )

    def wait_store(b):
      for s0 in range(8):
        pltpu.make_async_copy(
            tr[b].at[pl.ds(s0 * 2048, 2048)],
            out_hbm.at[pl.ds(s0 * 131072, 2048)],
            sem_s[b],
        ).wait()

    fire(job0, 0)

    @pl.loop(0, _JPW, step=2)
    def _outer(t0):
      for b in range(2):
        t = t0 + b
        nxt = t + 1

        @pl.when(nxt < _JPW)
        def _():
          @pl.when(nxt >= 2)
          def _():
            wait_store(1 - b)
          fire(job0 + nxt, 1 - b)

        wait_gathers(b)
        process(b)
        fire_store(job0 + t, b)

    wait_store(0)
    wait_store(1)

  return gather_kernel


_gather = _make_gather()


@jax.jit
def kernel(concept_indices, table):
  idx4d = concept_indices.T.reshape(_F, _NCH, 2, 128)
  table_p = jnp.pad(table, ((0, 0), (0, _D)))
  out1d = _gather(idx4d, table_p, jnp.asarray(_COLC_2D), jnp.asarray(_STC_2D))
  out5d = out1d.reshape(_F, 8, _NB // 128, 8, 128)
  return out5d.transpose(2, 4, 0, 1, 3).reshape(_NB, _F, _D)


# speedup vs baseline: 1.0375x; 1.0375x over previous
import functools

import numpy as np

import jax
import jax.numpy as jnp
from jax import lax
from jax.experimental import pallas as pl
from jax.experimental.pallas import tpu as pltpu
from jax.experimental.pallas import tpu_sc as plsc

_NB = 16384
_F = 26
_D = 64
_NC = 2
_NS = 16
_NW = _NC * _NS
_BC = 256
_NCH = _NB // _BC
_NJOB = _F * _NCH
_JPW = _NJOB // _NW
_NGRP = 3906
_V = 1000000

_COLC_NP = np.zeros((64, 16), np.int32)
_STC_NP = np.zeros((64, 16), np.int32)
_SRR_NP = np.zeros((64, 16), np.int32)
_SCA_NP = np.zeros((64, 16), np.int32)
for _m in range(16):
  for _k in range(4):
    for _l in range(16):
      _w = (_m + _l) & 15
      _ck = _m * 4 + _k
      _COLC_NP[_ck, _l] = _k * 16 + _w
      _STC_NP[_ck, _l] = (2 * _k + _w // 8) * 2048 + (_w % 8) * 128 + _l
      _SRR_NP[_ck, _l] = (2 * _k + _w // 8) * 16 + (_w % 8)
      _SCA_NP[_ck, _l] = (_l % 2) * 64 + _k * 16 + _w


def _make_relayout():
  mesh = plsc.VectorSubcoreMesh(core_axis_name="c", subcore_axis_name="s")

  @functools.partial(
      pl.kernel,
      out_type=jax.ShapeDtypeStruct((_V // 2, 128), jnp.float32),
      mesh=mesh,
      compiler_params=pltpu.CompilerParams(
          use_tc_tiling_on_sc=True, needs_layout_passes=False),
      scratch_types=[
          pltpu.VMEM((128, 128), jnp.float32),
          pltpu.VMEM((128, 128), jnp.float32),
          pltpu.VMEM((128, 128), jnp.float32),
          pltpu.VMEM((128, 128), jnp.float32),
          pltpu.VMEM((32, 128), jnp.float32),
          pltpu.VMEM((64, 16), jnp.int32),
          pltpu.VMEM((64, 16), jnp.int32),
          pltpu.SemaphoreType.DMA,
          pltpu.SemaphoreType.DMA,
          pltpu.SemaphoreType.DMA,
          pltpu.SemaphoreType.DMA,
      ],
  )
  def relayout_kernel(tabt_hbm, tail_hbm, srr_hbm, sca_hbm, out_hbm, inb0,
                      inb1, outb0, outb1, tailv, srr_v, sca_v, sem_i0, sem_i1,
                      sem_o0, sem_o1):
    wid = lax.axis_index("s") * _NC + lax.axis_index("c")
    pltpu.sync_copy(srr_hbm, srr_v)
    pltpu.sync_copy(sca_hbm, sca_v)
    inb = (inb0, inb1)
    outb = (outb0, outb1)
    sem_i = (sem_i0, sem_i1)
    sem_o = (sem_o0, sem_o1)
    g0 = wid * 122 + jnp.minimum(wid, 2)
    cnt = 122 + jnp.where(wid < 2, 1, 0)
    gend = g0 + cnt

    @pl.when(wid == 0)
    def _():
      pltpu.sync_copy(tail_hbm, tailv)
      pltpu.sync_copy(tailv, out_hbm.at[pl.ds(_V // 2 - 32, 32)])

    def fire(g, b):
      for q in range(2):
        for r in range(8):
          pltpu.async_copy(
              tabt_hbm.at[pl.ds(r * 8, 8), pl.ds((2 * g + q) * 128, 128)],
              inb[b].at[pl.ds((r * 2 + q) * 8, 8)],
              sem_i[b],
          )

    def wait_in(b):
      for _ in range(16):
        pltpu.make_async_copy(
            tabt_hbm.at[pl.ds(0, 8), pl.ds(0, 128)],
            inb[b].at[pl.ds(0, 8)],
            sem_i[b],
        ).wait()

    lanes = lax.iota(jnp.int32, 16)
    hlane = lanes // 2

    def process(b):
      @plsc.parallel_loop(0, 16, unroll=2)
      def _t(jb):
        lrow_s = (jb // 8) * 8
        lcol = lanes + lax.rem(jb, 8) * 16
        srow = jb * 8 + hlane
        pend = []
        for kk in range(4):
          for m in range(16):
            ck = m * 4 + kk
            vals = plsc.load_gather(inb[b], [srr_v[ck] + lrow_s, lcol])
            pend.append((ck, vals))
            if len(pend) >= 6:
              ck2, vals2 = pend.pop(0)
              plsc.store_scatter(outb[b], [srow, sca_v[ck2]], vals2)
        for ck2, vals2 in pend:
          plsc.store_scatter(outb[b], [srow, sca_v[ck2]], vals2)

    def fire_out(g, b):
      pltpu.async_copy(outb[b], out_hbm.at[pl.ds(g * 128, 128)], sem_o[b])

    def wait_out(b):
      pltpu.make_async_copy(
          outb[b], out_hbm.at[pl.ds(0, 128)], sem_o[b]).wait()

    fire(g0, 0)

    @pl.loop(0, 124, step=2)
    def _outer(t0):
      for b in range(2):
        g = g0 + t0 + b

        @pl.when(g < gend)
        def _():
          nxt = g + 1

          @pl.when(nxt < gend)
          def _():
            @pl.when(nxt >= g0 + 2)
            def _():
              wait_out(1 - b)
            fire(nxt, 1 - b)

          wait_in(b)
          process(b)
          fire_out(g, b)

    wait_out(0)
    wait_out(1)

  return relayout_kernel


def _make_gather():
  mesh = plsc.VectorSubcoreMesh(core_axis_name="c", subcore_axis_name="s")

  @functools.partial(
      pl.kernel,
      out_type=jax.ShapeDtypeStruct((_F * 1048576,), jnp.float32),
      mesh=mesh,
      compiler_params=pltpu.CompilerParams(
          use_tc_tiling_on_sc=False, needs_layout_passes=False),
      scratch_types=[
          pltpu.VMEM((2, 128), jnp.int32),
          pltpu.VMEM((2, 128), jnp.int32),
          pltpu.VMEM((2, 128), jnp.int32),
          pltpu.VMEM((2, 128), jnp.int32),
          pltpu.VMEM((256,), jnp.int32),
          pltpu.VMEM((256,), jnp.int32),
          pltpu.VMEM((_BC, 128), jnp.float32),
          pltpu.VMEM((_BC, 128), jnp.float32),
          pltpu.VMEM((16384,), jnp.float32),
          pltpu.VMEM((16384,), jnp.float32),
          pltpu.VMEM((64, 16), jnp.int32),
          pltpu.VMEM((64, 16), jnp.int32),
          pltpu.SemaphoreType.DMA,
          pltpu.SemaphoreType.DMA,
          pltpu.SemaphoreType.DMA,
          pltpu.SemaphoreType.DMA,
      ],
  )
  def gather_kernel(idx_hbm, table_hbm, colc_hbm, stc_hbm, out_hbm, idx_v0,
                    idx_v1, gidx_v0, gidx_v1, p_v0, p_v1, rows_v0, rows_v1,
                    tr_v0, tr_v1, colc_v, stc_v, sem_g0, sem_g1, sem_s0,
                    sem_s1):
    pltpu.sync_copy(colc_hbm, colc_v)
    pltpu.sync_copy(stc_hbm, stc_v)
    wid = lax.axis_index("s") * _NC + lax.axis_index("c")
    job0 = wid * _JPW
    idx_v = (idx_v0, idx_v1)
    gidx_v = (gidx_v0, gidx_v1)
    p_v = (p_v0, p_v1)
    rows = (rows_v0, rows_v1)
    tr = (tr_v0, tr_v1)
    sem_g = (sem_g0, sem_g1)
    sem_s = (sem_s0, sem_s1)

    def fire(j, b):
      f = j // _NCH
      c = lax.rem(j, _NCH)
      pltpu.sync_copy(idx_hbm.at[f, c], idx_v[b])
      for h in range(2):
        for k in range(8):
          v = idx_v[b][h, pl.ds(k * 16, 16)]
          gidx_v[b][h, pl.ds(k * 16, 16)] = lax.shift_right_logical(v, 1)
          p_v[b][pl.ds(h * 128 + k * 16, 16)] = lax.shift_left(
              lax.bitwise_and(v, 1), 6)
      for h in range(2):
        pltpu.async_copy(
            table_hbm.at[gidx_v[b].at[h]],
            rows[b].at[pl.ds(h * 128, 128)],
            sem_g[b],
        )

    def wait_gathers(b):
      for h in range(2):
        pltpu.make_async_copy(
            table_hbm.at[gidx_v[b].at[h]],
            rows[b].at[pl.ds(h * 128, 128)],
            sem_g[b],
        ).wait()

    lanes = lax.iota(jnp.int32, 16)

    def process(b):
      @plsc.parallel_loop(0, 16, unroll=2)
      def _trans(r16):
        rowvec = r16 * 16 + lanes
        pvec = p_v[b][pl.ds(r16 * 16, 16)]
        sb = jnp.zeros((16,), jnp.int32) + (
            (r16 // 8) * 1024 + lax.rem(r16, 8) * 16)
        pend = []
        for kk in range(4):
          for m in range(16):
            ck = m * 4 + kk
            vals = plsc.load_gather(rows[b], [rowvec, colc_v[ck] + pvec])
            pend.append((ck, vals))
            if len(pend) >= 6:
              ck2, vals2 = pend.pop(0)
              plsc.store_scatter(tr[b], [sb + stc_v[ck2]], vals2)
        for ck2, vals2 in pend:
          plsc.store_scatter(tr[b], [sb + stc_v[ck2]], vals2)

    def fire_store(j, b):
      f = j // _NCH
      c = lax.rem(j, _NCH)
      for s0 in range(8):
        pltpu.async_copy(
            tr[b].at[pl.ds(s0 * 2048, 2048)],
            out_hbm.at[pl.ds(f * 1048576 + s0 * 131072 + c * 2048, 2048)],
            sem_s[b],
        )

    def wait_store(b):
      for s0 in range(8):
        pltpu.make_async_copy(
            tr[b].at[pl.ds(s0 * 2048, 2048)],
            out_hbm.at[pl.ds(s0 * 131072, 2048)],
            sem_s[b],
        ).wait()

    fire(job0, 0)

    @pl.loop(0, _JPW, step=2)
    def _outer(t0):
      for b in range(2):
        t = t0 + b
        nxt = t + 1

        @pl.when(nxt < _JPW)
        def _():
          @pl.when(nxt >= 2)
          def _():
            wait_store(1 - b)
          fire(job0 + nxt, 1 - b)

        wait_gathers(b)
        process(b)
        fire_store(job0 + t, b)

    wait_store(0)
    wait_store(1)

  return gather_kernel


_relayout = _make_relayout()
_gather = _make_gather()


@jax.jit
def kernel(concept_indices, table):
  idx4d = concept_indices.T.reshape(_F, _NCH, 2, 128)
  tabt = table.T
  tail = table[_V - 64:].reshape(32, 128)
  pairs = _relayout(tabt, tail, jnp.asarray(_SRR_NP), jnp.asarray(_SCA_NP))
  out1d = _gather(idx4d, pairs, jnp.asarray(_COLC_NP), jnp.asarray(_STC_NP))
  out5d = out1d.reshape(_F, 8, _NB // 128, 8, 128)
  return out5d.transpose(2, 4, 0, 1, 3).reshape(_NB, _F, _D)


# speedup vs baseline: 1.0419x; 1.0043x over previous
import functools

import numpy as np

import jax
import jax.numpy as jnp
from jax import lax
from jax.experimental import pallas as pl
from jax.experimental.pallas import tpu as pltpu
from jax.experimental.pallas import tpu_sc as plsc

_NB = 16384
_F = 26
_D = 64
_NC = 2
_NS = 16
_NW = _NC * _NS
_BC = 256
_NCH = _NB // _BC
_NJOB = _F * _NCH
_JPW = _NJOB // _NW
_NGRP = 3906
_V = 1000000

_COLC_NP = np.zeros((64, 16), np.int32)
_STC_NP = np.zeros((64, 16), np.int32)
_SRR_NP = np.zeros((128, 16), np.int32)
_FSCA_NP = np.zeros((64, 16), np.int32)
for _m in range(16):
  for _k in range(4):
    for _l in range(16):
      _w = (_m + _l) & 15
      _ck = _m * 4 + _k
      _COLC_NP[_ck, _l] = _k * 16 + _w
      _STC_NP[_ck, _l] = (2 * _k + _w // 8) * 2048 + (_w % 8) * 128 + _l
      _SRR_NP[_ck, _l] = (2 * _k + _w // 8) * 16 + (_w % 8)
      _SRR_NP[64 + _ck, _l] = (2 * _k + _w // 8) * 16 + (_w % 8) + 8
      _FSCA_NP[_ck, _l] = _l * 64 + _k * 16 + _w


def _make_relayout():
  mesh = plsc.VectorSubcoreMesh(core_axis_name="c", subcore_axis_name="s")

  @functools.partial(
      pl.kernel,
      out_type=jax.ShapeDtypeStruct((_V * _D,), jnp.float32),
      mesh=mesh,
      compiler_params=pltpu.CompilerParams(
          use_tc_tiling_on_sc=True, needs_layout_passes=False),
      scratch_types=[
          pltpu.VMEM((128, 128), jnp.float32),
          pltpu.VMEM((128, 128), jnp.float32),
          pltpu.VMEM((16384,), jnp.float32),
          pltpu.VMEM((16384,), jnp.float32),
          pltpu.VMEM((4096,), jnp.float32),
          pltpu.VMEM((128, 16), jnp.int32),
          pltpu.VMEM((64, 16), jnp.int32),
          pltpu.SemaphoreType.DMA,
          pltpu.SemaphoreType.DMA,
          pltpu.SemaphoreType.DMA,
          pltpu.SemaphoreType.DMA,
      ],
  )
  def relayout_kernel(tabt_hbm, tail_hbm, srr_hbm, sca_hbm, out_hbm, inb0,
                      inb1, outb0, outb1, tailv, srr_v, sca_v, sem_i0, sem_i1,
                      sem_o0, sem_o1):
    wid = lax.axis_index("s") * _NC + lax.axis_index("c")
    pltpu.sync_copy(srr_hbm, srr_v)
    pltpu.sync_copy(sca_hbm, sca_v)
    inb = (inb0, inb1)
    outb = (outb0, outb1)
    sem_i = (sem_i0, sem_i1)
    sem_o = (sem_o0, sem_o1)
    g0 = wid * 122 + jnp.minimum(wid, 2)
    cnt = 122 + jnp.where(wid < 2, 1, 0)
    gend = g0 + cnt

    @pl.when(wid == 0)
    def _():
      pltpu.sync_copy(tail_hbm, tailv)
      pltpu.sync_copy(tailv, out_hbm.at[pl.ds(_V * _D - 4096, 4096)])

    def fire(g, b):
      for q in range(2):
        for r in range(8):
          pltpu.async_copy(
              tabt_hbm.at[pl.ds(r * 8, 8), pl.ds((2 * g + q) * 128, 128)],
              inb[b].at[pl.ds((r * 2 + q) * 8, 8)],
              sem_i[b],
          )

    def wait_in(b):
      for _ in range(16):
        pltpu.make_async_copy(
            tabt_hbm.at[pl.ds(0, 8), pl.ds(0, 128)],
            inb[b].at[pl.ds(0, 8)],
            sem_i[b],
        ).wait()

    lanes = lax.iota(jnp.int32, 16)

    def process(b):
      @plsc.parallel_loop(0, 16, unroll=2)
      def _t(jb):
        half = (jb // 8) * 64
        lcol = lanes + lax.rem(jb, 8) * 16
        sbase = jnp.zeros((16,), jnp.int32) + jb * 1024
        pend = []
        for kk in range(4):
          for m in range(16):
            ck = m * 4 + kk
            vals = plsc.load_gather(inb[b], [srr_v[half + ck], lcol])
            pend.append((ck, vals))
            if len(pend) >= 6:
              ck2, vals2 = pend.pop(0)
              plsc.store_scatter(outb[b], [sbase + sca_v[ck2]], vals2)
        for ck2, vals2 in pend:
          plsc.store_scatter(outb[b], [sbase + sca_v[ck2]], vals2)

    def fire_out(g, b):
      pltpu.async_copy(
          outb[b], out_hbm.at[pl.ds(g * 16384, 16384)], sem_o[b])

    def wait_out(b):
      pltpu.make_async_copy(
          outb[b], out_hbm.at[pl.ds(0, 16384)], sem_o[b]).wait()

    fire(g0, 0)

    @pl.loop(0, 124, step=2)
    def _outer(t0):
      for b in range(2):
        g = g0 + t0 + b

        @pl.when(g < gend)
        def _():
          nxt = g + 1

          @pl.when(nxt < gend)
          def _():
            @pl.when(nxt >= g0 + 2)
            def _():
              wait_out(1 - b)
            fire(nxt, 1 - b)

          wait_in(b)
          process(b)
          fire_out(g, b)

    wait_out(0)
    wait_out(1)

  return relayout_kernel


def _make_gather():
  mesh = plsc.VectorSubcoreMesh(core_axis_name="c", subcore_axis_name="s")

  @functools.partial(
      pl.kernel,
      out_type=jax.ShapeDtypeStruct((_F * 1048576,), jnp.float32),
      mesh=mesh,
      compiler_params=pltpu.CompilerParams(
          use_tc_tiling_on_sc=False, needs_layout_passes=False),
      scratch_types=[
          pltpu.VMEM((2, 128), jnp.int32),
          pltpu.VMEM((2, 128), jnp.int32),
          pltpu.VMEM((_BC, _D), jnp.float32),
          pltpu.VMEM((_BC, _D), jnp.float32),
          pltpu.VMEM((16384,), jnp.float32),
          pltpu.VMEM((16384,), jnp.float32),
          pltpu.VMEM((64, 16), jnp.int32),
          pltpu.VMEM((64, 16), jnp.int32),
          pltpu.SemaphoreType.DMA,
          pltpu.SemaphoreType.DMA,
          pltpu.SemaphoreType.DMA,
          pltpu.SemaphoreType.DMA,
      ],
  )
  def gather_kernel(idx_hbm, table_hbm, colc_hbm, stc_hbm, out_hbm, idx_v0,
                    idx_v1, rows_v0, rows_v1, tr_v0, tr_v1, colc_v, stc_v,
                    sem_g0, sem_g1, sem_s0, sem_s1):
    pltpu.sync_copy(colc_hbm, colc_v)
    pltpu.sync_copy(stc_hbm, stc_v)
    wid = lax.axis_index("s") * _NC + lax.axis_index("c")
    job0 = wid * _JPW
    idx_v = (idx_v0, idx_v1)
    rows = (rows_v0, rows_v1)
    tr = (tr_v0, tr_v1)
    sem_g = (sem_g0, sem_g1)
    sem_s = (sem_s0, sem_s1)

    def fire(j, b):
      f = j // _NCH
      c = lax.rem(j, _NCH)
      pltpu.sync_copy(idx_hbm.at[f, c], idx_v[b])
      for h in range(2):
        pltpu.async_copy(
            table_hbm.at[idx_v[b].at[h]],
            rows[b].at[pl.ds(h * 128, 128)],
            sem_g[b],
        )

    def wait_gathers(b):
      for h in range(2):
        pltpu.make_async_copy(
            table_hbm.at[idx_v[b].at[h]],
            rows[b].at[pl.ds(h * 128, 128)],
            sem_g[b],
        ).wait()

    lanes = lax.iota(jnp.int32, 16)

    def process(b):
      @plsc.parallel_loop(0, 16, unroll=2)
      def _trans(r16):
        rowvec = r16 * 16 + lanes
        sb = jnp.zeros((16,), jnp.int32) + (
            (r16 // 8) * 1024 + lax.rem(r16, 8) * 16)
        pend = []
        for kk in range(4):
          for m in range(16):
            ck = m * 4 + kk
            vals = plsc.load_gather(rows[b], [rowvec, colc_v[ck]])
            pend.append((ck, vals))
            if len(pend) >= 6:
              ck2, vals2 = pend.pop(0)
              plsc.store_scatter(tr[b], [sb + stc_v[ck2]], vals2)
        for ck2, vals2 in pend:
          plsc.store_scatter(tr[b], [sb + stc_v[ck2]], vals2)

    def fire_store(j, b):
      f = j // _NCH
      c = lax.rem(j, _NCH)
      for s0 in range(8):
        pltpu.async_copy(
            tr[b].at[pl.ds(s0 * 2048, 2048)],
            out_hbm.at[pl.ds(f * 1048576 + s0 * 131072 + c * 2048, 2048)],
            sem_s[b],
        )

    def wait_store(b):
      for s0 in range(8):
        pltpu.make_async_copy(
            tr[b].at[pl.ds(s0 * 2048, 2048)],
            out_hbm.at[pl.ds(s0 * 131072, 2048)],
            sem_s[b],
        ).wait()

    fire(job0, 0)

    @pl.loop(0, _JPW, step=2)
    def _outer(t0):
      for b in range(2):
        t = t0 + b
        nxt = t + 1

        @pl.when(nxt < _JPW)
        def _():
          @pl.when(nxt >= 2)
          def _():
            wait_store(1 - b)
          fire(job0 + nxt, 1 - b)

        wait_gathers(b)
        process(b)
        fire_store(job0 + t, b)

    wait_store(0)
    wait_store(1)

  return gather_kernel


_relayout = _make_relayout()
_gather = _make_gather()


@jax.jit
def kernel(concept_indices, table):
  idx4d = concept_indices.T.reshape(_F, _NCH, 2, 128)
  tabt = table.T
  tail = table[_V - 64:].reshape(-1)
  tlin = _relayout(tabt, tail, jnp.asarray(_SRR_NP), jnp.asarray(_FSCA_NP))
  out1d = _gather(idx4d, tlin.reshape(_V, _D), jnp.asarray(_COLC_NP),
                  jnp.asarray(_STC_NP))
  out5d = out1d.reshape(_F, 8, _NB // 128, 8, 128)
  return out5d.transpose(2, 4, 0, 1, 3).reshape(_NB, _F, _D)


# speedup vs baseline: 1.0496x; 1.0074x over previous
import functools

import numpy as np

import jax
import jax.numpy as jnp
from jax import lax
from jax.experimental import pallas as pl
from jax.experimental.pallas import tpu as pltpu
from jax.experimental.pallas import tpu_sc as plsc

_NB = 16384
_F = 26
_D = 64
_NC = 2
_NS = 16
_NW = _NC * _NS
_BC = 256
_NCH = _NB // _BC
_NJOB = _F * _NCH
_JPW = _NJOB // _NW
_NGRP = 3906
_V = 1000000

_COLC_NP = np.zeros((64, 16), np.int32)
_STC_NP = np.zeros((64, 16), np.int32)
_SRR_NP = np.zeros((128, 16), np.int32)
_FSCA_NP = np.zeros((64, 16), np.int32)
for _m in range(16):
  for _k in range(4):
    for _l in range(16):
      _w = (_m + _l) & 15
      _ck = _m * 4 + _k
      _COLC_NP[_ck, _l] = _k * 16 + _w
      _STC_NP[_ck, _l] = (2 * _k + _w // 8) * 2048 + (_w % 8) * 128 + _l
      _SRR_NP[_ck, _l] = (2 * _k + _w // 8) * 16 + (_w % 8)
      _SRR_NP[64 + _ck, _l] = (2 * _k + _w // 8) * 16 + (_w % 8) + 8
      _FSCA_NP[_ck, _l] = _l * 64 + _k * 16 + _w


def _make_relayout():
  mesh = plsc.VectorSubcoreMesh(core_axis_name="c", subcore_axis_name="s")

  @functools.partial(
      pl.kernel,
      out_type=jax.ShapeDtypeStruct((_V * _D,), jnp.float32),
      mesh=mesh,
      compiler_params=pltpu.CompilerParams(
          use_tc_tiling_on_sc=True, needs_layout_passes=False),
      scratch_types=[
          pltpu.VMEM((64, 256), jnp.float32),
          pltpu.VMEM((64, 256), jnp.float32),
          pltpu.VMEM((16384,), jnp.float32),
          pltpu.VMEM((16384,), jnp.float32),
          pltpu.VMEM((4096,), jnp.float32),
          pltpu.VMEM((64, 16), jnp.int32),
          pltpu.VMEM((64, 16), jnp.int32),
          pltpu.SemaphoreType.DMA,
          pltpu.SemaphoreType.DMA,
          pltpu.SemaphoreType.DMA,
          pltpu.SemaphoreType.DMA,
      ],
  )
  def relayout_kernel(tabt_hbm, tail_hbm, srr_hbm, sca_hbm, out_hbm, inb0,
                      inb1, outb0, outb1, tailv, srr_v, sca_v, sem_i0, sem_i1,
                      sem_o0, sem_o1):
    wid = lax.axis_index("s") * _NC + lax.axis_index("c")
    pltpu.sync_copy(srr_hbm, srr_v)
    pltpu.sync_copy(sca_hbm, sca_v)
    inb = (inb0, inb1)
    outb = (outb0, outb1)
    sem_i = (sem_i0, sem_i1)
    sem_o = (sem_o0, sem_o1)
    g0 = wid * 122 + jnp.minimum(wid, 2)
    cnt = 122 + jnp.where(wid < 2, 1, 0)
    gend = g0 + cnt

    @pl.when(wid == 0)
    def _():
      pltpu.sync_copy(tail_hbm, tailv)
      pltpu.sync_copy(tailv, out_hbm.at[pl.ds(_V * _D - 4096, 4096)])

    def fire(g, b):
      pltpu.async_copy(
          tabt_hbm.at[pl.ds(0, 64), pl.ds(g * 256, 256)],
          inb[b],
          sem_i[b],
      )

    def wait_in(b):
      pltpu.make_async_copy(
          tabt_hbm.at[pl.ds(0, 64), pl.ds(0, 256)],
          inb[b],
          sem_i[b],
      ).wait()

    lanes = lax.iota(jnp.int32, 16)

    def process(b):
      @plsc.parallel_loop(0, 16, unroll=2)
      def _t(jb):
        jvec = lanes + jb * 16
        sbase = jnp.zeros((16,), jnp.int32) + jb * 1024
        pend = []
        for kk in range(4):
          for m in range(16):
            ck = m * 4 + kk
            vals = plsc.load_gather(inb[b], [srr_v[ck], jvec])
            pend.append((ck, vals))
            if len(pend) >= 6:
              ck2, vals2 = pend.pop(0)
              plsc.store_scatter(outb[b], [sbase + sca_v[ck2]], vals2)
        for ck2, vals2 in pend:
          plsc.store_scatter(outb[b], [sbase + sca_v[ck2]], vals2)

    def fire_out(g, b):
      pltpu.async_copy(
          outb[b], out_hbm.at[pl.ds(g * 16384, 16384)], sem_o[b])

    def wait_out(b):
      pltpu.make_async_copy(
          outb[b], out_hbm.at[pl.ds(0, 16384)], sem_o[b]).wait()

    fire(g0, 0)

    @pl.loop(0, 124, step=2)
    def _outer(t0):
      for b in range(2):
        g = g0 + t0 + b

        @pl.when(g < gend)
        def _():
          nxt = g + 1

          @pl.when(nxt < gend)
          def _():
            @pl.when(nxt >= g0 + 2)
            def _():
              wait_out(1 - b)
            fire(nxt, 1 - b)

          wait_in(b)
          process(b)
          fire_out(g, b)

    wait_out(0)
    wait_out(1)

  return relayout_kernel


def _make_gather():
  mesh = plsc.VectorSubcoreMesh(core_axis_name="c", subcore_axis_name="s")

  @functools.partial(
      pl.kernel,
      out_type=jax.ShapeDtypeStruct((_F * 1048576,), jnp.float32),
      mesh=mesh,
      compiler_params=pltpu.CompilerParams(
          use_tc_tiling_on_sc=False, needs_layout_passes=False),
      scratch_types=[
          pltpu.VMEM((2, 128), jnp.int32),
          pltpu.VMEM((2, 128), jnp.int32),
          pltpu.VMEM((_BC, _D), jnp.float32),
          pltpu.VMEM((_BC, _D), jnp.float32),
          pltpu.VMEM((16384,), jnp.float32),
          pltpu.VMEM((16384,), jnp.float32),
          pltpu.VMEM((64, 16), jnp.int32),
          pltpu.VMEM((64, 16), jnp.int32),
          pltpu.SemaphoreType.DMA,
          pltpu.SemaphoreType.DMA,
          pltpu.SemaphoreType.DMA,
          pltpu.SemaphoreType.DMA,
      ],
  )
  def gather_kernel(idx_hbm, table_hbm, colc_hbm, stc_hbm, out_hbm, idx_v0,
                    idx_v1, rows_v0, rows_v1, tr_v0, tr_v1, colc_v, stc_v,
                    sem_g0, sem_g1, sem_s0, sem_s1):
    pltpu.sync_copy(colc_hbm, colc_v)
    pltpu.sync_copy(stc_hbm, stc_v)
    wid = lax.axis_index("s") * _NC + lax.axis_index("c")
    job0 = wid * _JPW
    idx_v = (idx_v0, idx_v1)
    rows = (rows_v0, rows_v1)
    tr = (tr_v0, tr_v1)
    sem_g = (sem_g0, sem_g1)
    sem_s = (sem_s0, sem_s1)

    def fire(j, b):
      f = j // _NCH
      c = lax.rem(j, _NCH)
      pltpu.sync_copy(idx_hbm.at[f, c], idx_v[b])
      for h in range(2):
        pltpu.async_copy(
            table_hbm.at[idx_v[b].at[h]],
            rows[b].at[pl.ds(h * 128, 128)],
            sem_g[b],
        )

    def wait_gathers(b):
      for h in range(2):
        pltpu.make_async_copy(
            table_hbm.at[idx_v[b].at[h]],
            rows[b].at[pl.ds(h * 128, 128)],
            sem_g[b],
        ).wait()

    lanes = lax.iota(jnp.int32, 16)

    def process(b):
      @plsc.parallel_loop(0, 16, unroll=2)
      def _trans(r16):
        rowvec = r16 * 16 + lanes
        sb = jnp.zeros((16,), jnp.int32) + (
            (r16 // 8) * 1024 + lax.rem(r16, 8) * 16)
        pend = []
        for kk in range(4):
          for m in range(16):
            ck = m * 4 + kk
            vals = plsc.load_gather(rows[b], [rowvec, colc_v[ck]])
            pend.append((ck, vals))
            if len(pend) >= 6:
              ck2, vals2 = pend.pop(0)
              plsc.store_scatter(tr[b], [sb + stc_v[ck2]], vals2)
        for ck2, vals2 in pend:
          plsc.store_scatter(tr[b], [sb + stc_v[ck2]], vals2)

    def fire_store(j, b):
      f = j // _NCH
      c = lax.rem(j, _NCH)
      for s0 in range(8):
        pltpu.async_copy(
            tr[b].at[pl.ds(s0 * 2048, 2048)],
            out_hbm.at[pl.ds(f * 1048576 + s0 * 131072 + c * 2048, 2048)],
            sem_s[b],
        )

    def wait_store(b):
      for s0 in range(8):
        pltpu.make_async_copy(
            tr[b].at[pl.ds(s0 * 2048, 2048)],
            out_hbm.at[pl.ds(s0 * 131072, 2048)],
            sem_s[b],
        ).wait()

    fire(job0, 0)

    @pl.loop(0, _JPW, step=2)
    def _outer(t0):
      for b in range(2):
        t = t0 + b
        nxt = t + 1

        @pl.when(nxt < _JPW)
        def _():
          @pl.when(nxt >= 2)
          def _():
            wait_store(1 - b)
          fire(job0 + nxt, 1 - b)

        wait_gathers(b)
        process(b)
        fire_store(job0 + t, b)

    wait_store(0)
    wait_store(1)

  return gather_kernel


_relayout = _make_relayout()
_gather = _make_gather()


@jax.jit
def kernel(concept_indices, table):
  idx4d = concept_indices.T.reshape(_F, _NCH, 2, 128)
  tabt = table.T
  tail = table[_V - 64:].reshape(-1)
  tlin = _relayout(tabt, tail, jnp.asarray(_COLC_NP), jnp.asarray(_FSCA_NP))
  out1d = _gather(idx4d, tlin.reshape(_V, _D), jnp.asarray(_COLC_NP),
                  jnp.asarray(_STC_NP))
  out5d = out1d.reshape(_F, 8, _NB // 128, 8, 128)
  return out5d.transpose(2, 4, 0, 1, 3).reshape(_NB, _F, _D)


# speedup vs baseline: 1.1920x; 1.1357x over previous
import functools

import numpy as np

import jax
import jax.numpy as jnp
from jax import lax
from jax.experimental import pallas as pl
from jax.experimental.pallas import tpu as pltpu
from jax.experimental.pallas import tpu_sc as plsc

_NB = 16384
_F = 26
_D = 64
_NC = 2
_NS = 16
_NW = _NC * _NS
_BC = 256
_NCH = _NB // _BC
_NJOB = _F * _NCH
_JPW = _NJOB // _NW


_COLC_NP = np.zeros((16, 4, 16), np.int32)
_STC_NP = np.zeros((16, 4, 16), np.int32)
for _m in range(16):
  for _k in range(4):
    for _l in range(16):
      _w = (_m + _l) & 15
      _COLC_NP[_m, _k, _l] = _k * 16 + _w
      _STC_NP[_m, _k, _l] = (2 * _k + _w // 8) * 2048 + (_w % 8) * 128 + _l
_COLC_2D = _COLC_NP.reshape(64, 16)
_STC_2D = _STC_NP.reshape(64, 16)


def _make_gather():
  mesh = plsc.VectorSubcoreMesh(core_axis_name="c", subcore_axis_name="s")

  @functools.partial(
      pl.kernel,
      out_type=jax.ShapeDtypeStruct((_F * 1048576,), jnp.float32),
      mesh=mesh,
      compiler_params=pltpu.CompilerParams(
          use_tc_tiling_on_sc=False, needs_layout_passes=False),
      scratch_types=[
          pltpu.VMEM((2, 128), jnp.int32),
          pltpu.VMEM((2, 128), jnp.int32),
          pltpu.VMEM((_BC, 128), jnp.float32),
          pltpu.VMEM((_BC, 128), jnp.float32),
          pltpu.VMEM((16384,), jnp.float32),
          pltpu.VMEM((16384,), jnp.float32),
          pltpu.VMEM((64, 16), jnp.int32),
          pltpu.VMEM((64, 16), jnp.int32),
          pltpu.SemaphoreType.DMA,
          pltpu.SemaphoreType.DMA,
          pltpu.SemaphoreType.DMA,
          pltpu.SemaphoreType.DMA,
      ],
  )
  def gather_kernel(idx_hbm, table_hbm, colc_hbm, stc_hbm, out_hbm, idx_v0,
                    idx_v1, rows_v0, rows_v1, tr_v0, tr_v1, colc_v, stc_v,
                    sem_g0, sem_g1, sem_s0, sem_s1):
    pltpu.sync_copy(colc_hbm, colc_v)
    pltpu.sync_copy(stc_hbm, stc_v)
    wid = lax.axis_index("s") * _NC + lax.axis_index("c")
    job0 = wid * _JPW
    idx_v = (idx_v0, idx_v1)
    rows = (rows_v0, rows_v1)
    tr = (tr_v0, tr_v1)
    sem_g = (sem_g0, sem_g1)
    sem_s = (sem_s0, sem_s1)

    def fire(j, b):
      f = j // _NCH
      c = lax.rem(j, _NCH)
      pltpu.sync_copy(idx_hbm.at[f, c], idx_v[b])
      for h in range(2):
        pltpu.async_copy(
            table_hbm.at[idx_v[b].at[h]],
            rows[b].at[pl.ds(h * 128, 128)],
            sem_g[b],
        )

    def wait_gathers(b):
      for h in range(2):
        pltpu.make_async_copy(
            table_hbm.at[idx_v[b].at[h]],
            rows[b].at[pl.ds(h * 128, 128)],
            sem_g[b],
        ).wait()

    def process(b):
      lanes = lax.iota(jnp.int32, 16)

      @plsc.parallel_loop(0, 16, unroll=2)
      def _trans(r16):
        rowvec = r16 * 16 + lanes
        sb = jnp.zeros((16,), jnp.int32) + ((r16 // 8) * 1024 + lax.rem(r16, 8) * 16)
        pend = []
        for kk in range(4):
          for m in range(16):
            ck = m * 4 + kk
            vals = plsc.load_gather(rows[b], [rowvec, colc_v[ck]])
            pend.append((ck, vals))
            if len(pend) >= 6:
              ck2, vals2 = pend.pop(0)
              plsc.store_scatter(tr[b], [sb + stc_v[ck2]], vals2)
        for ck2, vals2 in pend:
          plsc.store_scatter(tr[b], [sb + stc_v[ck2]], vals2)

    def fire_store(j, b):
      f = j // _NCH
      c = lax.rem(j, _NCH)
      for s0 in range(8):
        pltpu.async_copy(
            tr[b].at[pl.ds(s0 * 2048, 2048)],
            out_hbm.at[pl.ds(f * 1048576 + s0 * 131072 + c * 2048, 2048)],
            sem_s[b],
        )

    def wait_store(b):
      for s0 in range(8):
        pltpu.make_async_copy(
            tr[b].at[pl.ds(s0 * 2048, 2048)],
            out_hbm.at[pl.ds(s0 * 131072, 2048)],
            sem_s[b],
        ).wait()

    fire(job0, 0)

    @pl.loop(0, _JPW, step=2)
    def _outer(t0):
      for b in range(2):
        t = t0 + b
        nxt = t + 1

        @pl.when(nxt < _JPW)
        def _():
          @pl.when(nxt >= 2)
          def _():
            wait_store(1 - b)
          fire(job0 + nxt, 1 - b)

        wait_gathers(b)
        process(b)
        fire_store(job0 + t, b)

    wait_store(0)
    wait_store(1)

  return gather_kernel


_gather = _make_gather()


@jax.jit
def kernel(concept_indices, table):
  idx4d = concept_indices.T.reshape(_F, _NCH, 2, 128)
  table_p = jnp.pad(table, ((0, 0), (0, _D)))
  out1d = _gather(idx4d, table_p, jnp.asarray(_COLC_2D), jnp.asarray(_STC_2D))
  out5d = out1d.reshape(_F, 8, _NB // 128, 8, 128)
  return out5d.transpose(2, 4, 0, 1, 3).reshape(_NB, _F, _D)
